# baseline (speedup 1.0000x reference)
import jax
import jax.numpy as jnp
from jax import lax
from jax.experimental import pallas as pl
from jax.experimental.pallas import tpu as pltpu
from jax.experimental.pallas import tpu_sc as plsc

DIM = 64
BATCH = 16384
CTX = 20

_NC = 2
_NS = 16
_NW = _NC * _NS
_BPW = BATCH // _NW
_GB = 16
_NG = _BPW // _GB
_ROWS = _GB * CTX
_CH = 64
_NCH = _ROWS // _CH


def _sc_body(target_hbm, ctxidx_hbm, skip_hbm, ctxtab_hbm, out_hbm,
             tidx_v, cidx_v, tgt_v, ctx_v, out_v, sem):
    wid = lax.axis_index("s") * _NC + lax.axis_index("c")
    iota = lax.broadcasted_iota(jnp.int32, (16,), 0)
    iota_ctx = iota * CTX
    iota_dim = iota * DIM
    iota_cd = iota * (CTX * DIM)

    def group(g, carry):
        b0 = wid * _BPW + g * _GB
        p0 = b0 * CTX
        pltpu.sync_copy(target_hbm.at[pl.ds(b0, _GB)], tidx_v)
        pltpu.sync_copy(ctxidx_hbm.at[pl.ds(p0, _ROWS)], cidx_v)
        cps = [pltpu.async_copy(skip_hbm.at[tidx_v], tgt_v, sem)]
        for ch in range(_NCH):
            cps.append(pltpu.async_copy(
                ctxtab_hbm.at[cidx_v.at[pl.ds(ch * _CH, _CH)]],
                ctx_v.at[pl.ds(ch * _CH, _CH)], sem))
        for cp in cps:
            cp.wait()

        def dstep(d, accs):
            col = jnp.full((16,), d, jnp.int32)
            tcol = plsc.load_gather(tgt_v, [iota, col])
            return tuple(
                accs[c] + tcol * plsc.load_gather(ctx_v, [iota_ctx + c, col])
                for c in range(CTX))

        accs = lax.fori_loop(
            0, DIM, dstep,
            tuple(jnp.zeros((16,), jnp.float32) for _ in range(CTX)))
        for c in range(CTX):
            plsc.store_scatter(out_v, [iota_ctx + c], accs[c])
        pltpu.sync_copy(out_v, out_hbm.at[pl.ds(p0, _ROWS)])
        return carry

    lax.fori_loop(0, _NG, group, 0)


def kernel(target, context, skipgram_table, context_table):
    mesh = plsc.VectorSubcoreMesh(core_axis_name="c", subcore_axis_name="s")
    f = pl.kernel(
        _sc_body,
        out_type=jax.ShapeDtypeStruct((BATCH * CTX,), jnp.float32),
        mesh=mesh,
        scratch_types=[
            pltpu.VMEM((_GB,), jnp.int32),
            pltpu.VMEM((_ROWS,), jnp.int32),
            pltpu.VMEM((_GB, DIM), jnp.float32),
            pltpu.VMEM((_ROWS, DIM), jnp.float32),
            pltpu.VMEM((_ROWS,), jnp.float32),
            pltpu.SemaphoreType.DMA,
        ],
        compiler_params=pltpu.CompilerParams(
            needs_layout_passes=False, use_tc_tiling_on_sc=False),
    )
    out = f(target.astype(jnp.int32), context.reshape(-1).astype(jnp.int32),
            skipgram_table, context_table)
    return out.reshape(BATCH, CTX)
